# baseline (speedup 1.0000x reference)
import functools

import jax
import jax.numpy as jnp
from jax import lax
from jax.experimental import pallas as pl
from jax.experimental.pallas import tpu as pltpu
from jax.experimental.pallas import tpu_sc as plsc

T = 2048
D = 1024
E = 8
K = 2
I_R = 512
I_S = 1024
ALPHA = 0.0001
BLK = 256
G = 24
NPAD = G * BLK
NEG = -3.0e38

NC, NS = 2, 16
NW = NC * NS


def _sigmoid(v):
    return 1.0 / (1.0 + jnp.exp(-v))


def _router_body(xf_ref, rw_ref, bias_ref, p1_ref, p2_ref, g1_ref, g2_ref,
                 cnt_ref, loss_ref, be_ref):
    xf = xf_ref[...]
    logits = jnp.dot(xf, rw_ref[...], preferred_element_type=jnp.float32)
    aff = _sigmoid(logits)
    scores = aff + bias_ref[...]
    iota_e = lax.broadcasted_iota(jnp.int32, (T, E), 1)

    m1 = jnp.max(scores, axis=1, keepdims=True)
    i1 = jnp.min(jnp.where(scores == m1, iota_e, E), axis=1, keepdims=True)
    sel1 = iota_e == i1
    masked = jnp.where(sel1, NEG, scores)
    m2 = jnp.max(masked, axis=1, keepdims=True)
    i2 = jnp.min(jnp.where(masked == m2, iota_e, E), axis=1, keepdims=True)
    sel2 = iota_e == i2

    a1 = jnp.sum(jnp.where(sel1, aff, 0.0), axis=1, keepdims=True)
    a2 = jnp.sum(jnp.where(sel2, aff, 0.0), axis=1, keepdims=True)
    den = a1 + a2 + 1e-9
    g1_ref[...] = a1 / den
    g2_ref[...] = a2 / den

    mask = jnp.where(sel1 | sel2, 1.0, 0.0)
    counts_f = jnp.sum(mask, axis=0, keepdims=True)
    cnt_ref[...] = counts_f.astype(jnp.int32)

    rowsum = jnp.sum(aff, axis=1, keepdims=True) + 1e-9
    p_mean = jnp.sum(aff / rowsum, axis=0, keepdims=True) * (1.0 / T)
    f_bal = counts_f * (E / (K * T))
    loss_ref[...] = jnp.sum(f_bal * p_mean, axis=1, keepdims=True) * ALPHA

    r_i = lax.broadcasted_iota(jnp.int32, (BLK, BLK), 0)
    c_i = lax.broadcasted_iota(jnp.int32, (BLK, BLK), 1)
    tri = jnp.where(r_i >= c_i, 1.0, 0.0)
    nblk = T // BLK
    parts = []
    prefix = jnp.zeros((1, E), jnp.float32)
    for b in range(nblk):
        blk = lax.slice(mask, (b * BLK, 0), ((b + 1) * BLK, E))
        within = jnp.dot(tri, blk, preferred_element_type=jnp.float32)
        parts.append(within + prefix)
        prefix = prefix + lax.slice(within, (BLK - 1, 0), (BLK, E))
    rank = jnp.concatenate(parts, axis=0)

    nb_e = jnp.floor((counts_f + (BLK - 1)) * (1.0 / BLK))
    pc = nb_e * BLK
    s_r = lax.broadcasted_iota(jnp.int32, (E, E), 0)
    s_c = lax.broadcasted_iota(jnp.int32, (E, E), 1)
    strict = jnp.where(s_r < s_c, 1.0, 0.0)
    off = jnp.dot(pc, strict, preferred_element_type=jnp.float32)

    off_b = jnp.broadcast_to(off, (T, E))
    o1 = jnp.sum(jnp.where(sel1, off_b, 0.0), axis=1, keepdims=True)
    o2 = jnp.sum(jnp.where(sel2, off_b, 0.0), axis=1, keepdims=True)
    r1 = jnp.sum(jnp.where(sel1, rank, 0.0), axis=1, keepdims=True)
    r2 = jnp.sum(jnp.where(sel2, rank, 0.0), axis=1, keepdims=True)
    p1_ref[...] = (o1 + r1 - 1.0).astype(jnp.int32)
    p2_ref[...] = (o2 + r2 - 1.0).astype(jnp.int32)

    ends = off + pc
    g_f = lax.broadcasted_iota(jnp.int32, (1, 32), 1).astype(jnp.float32) * float(BLK)
    lane8 = lax.broadcasted_iota(jnp.int32, (1, E), 1)
    be = jnp.zeros((1, 32), jnp.float32)
    for e in range(E):
        end_e = jnp.sum(jnp.where(lane8 == e, ends, 0.0), axis=1, keepdims=True)
        be = be + jnp.where(end_e <= g_f, 1.0, 0.0)
    be_ref[...] = jnp.minimum(be, E - 1.0).astype(jnp.int32)


def _router(xf, router_w, expert_bias):
    return pl.pallas_call(
        _router_body,
        out_shape=(
            jax.ShapeDtypeStruct((T, 1), jnp.int32),
            jax.ShapeDtypeStruct((T, 1), jnp.int32),
            jax.ShapeDtypeStruct((T, 1), jnp.float32),
            jax.ShapeDtypeStruct((T, 1), jnp.float32),
            jax.ShapeDtypeStruct((1, E), jnp.int32),
            jax.ShapeDtypeStruct((1, 1), jnp.float32),
            jax.ShapeDtypeStruct((1, 32), jnp.int32),
        ),
    )(xf, router_w, expert_bias.reshape(1, E))


def _build_inv_body(p1_h, p2_h, g1_h, g2_h, itok_h, igate_h, it_v, ig_v, pv, gv):
    wid = lax.axis_index("s") * NC + lax.axis_index("c")

    @pl.when(wid == 0)
    def _():
        def zero_body(i, c):
            it_v[pl.ds(i * 16, 16)] = jnp.zeros((16,), jnp.int32)
            ig_v[pl.ds(i * 16, 16)] = jnp.zeros((16,), jnp.float32)
            return c
        lax.fori_loop(0, NPAD // 16, zero_body, 0)
        for p_h, g_h in ((p1_h, g1_h), (p2_h, g2_h)):
            pltpu.sync_copy(p_h, pv)
            pltpu.sync_copy(g_h, gv)

            def scat_body(i, c):
                sl = pl.ds(i * 16, 16)
                idx = pv[sl]
                toks = lax.iota(jnp.int32, 16) + i * 16
                plsc.store_scatter(it_v, [idx], toks)
                plsc.store_scatter(ig_v, [idx], gv[sl])
                return c
            lax.fori_loop(0, T // 16, scat_body, 0)
        pltpu.sync_copy(it_v, itok_h)
        pltpu.sync_copy(ig_v, igate_h)


def _build_inv(p1, p2, g1, g2):
    mesh = plsc.VectorSubcoreMesh(core_axis_name="c", subcore_axis_name="s")
    fn = pl.kernel(
        _build_inv_body,
        mesh=mesh,
        compiler_params=pltpu.CompilerParams(needs_layout_passes=False),
        out_type=(
            jax.ShapeDtypeStruct((NPAD,), jnp.int32),
            jax.ShapeDtypeStruct((NPAD,), jnp.float32),
        ),
        scratch_types=[
            pltpu.VMEM((NPAD,), jnp.int32),
            pltpu.VMEM((NPAD,), jnp.float32),
            pltpu.VMEM((T,), jnp.int32),
            pltpu.VMEM((T,), jnp.float32),
        ],
    )
    return fn(p1, p2, g1, g2)


_GNB = 4
_GRP = (NPAD // NW) // 8


def _gather_body(itok_h, xflat_h, xg_h, idx_v, *rest):
    bufs = rest[:_GNB]
    gsems = rest[_GNB:2 * _GNB]
    wsems = rest[2 * _GNB:3 * _GNB]
    wid = lax.axis_index("s") * NC + lax.axis_index("c")
    rows_per_w = NPAD // NW
    base = wid * rows_per_w
    pltpu.sync_copy(itok_h.at[pl.ds(base, rows_per_w)], idx_v)
    iota16 = lax.iota(jnp.int32, 16)
    gcp = [None] * _GRP
    wcp = [None] * _GRP

    def start_group(g):
        b = g % _GNB
        vec = idx_v[pl.ds((g // 2) * 16, 16)]
        cps = []
        for r in range(8):
            lane = (g % 2) * 8 + r
            tok = jnp.sum(jnp.where(iota16 == lane, vec, 0))
            cps.append(pltpu.async_copy(
                xflat_h.at[pl.ds(tok * D, D)], bufs[b].at[r], gsems[b]))
        gcp[g] = cps

    def start_write(g):
        b = g % _GNB
        wcp[g] = pltpu.async_copy(
            bufs[b], xg_h.at[pl.ds(base + g * 8, 8)], wsems[b])

    for g in range(_GNB):
        start_group(g)
    for g in range(_GRP):
        for cp in gcp[g]:
            cp.wait()
        start_write(g)
        nxt = g + _GNB
        if nxt < _GRP:
            wcp[g].wait()
            start_group(nxt)
    for g in range(_GRP - _GNB, _GRP):
        wcp[g].wait()


def _gather_rows(inv_tok, xflat):
    mesh = plsc.VectorSubcoreMesh(core_axis_name="c", subcore_axis_name="s")
    fn = pl.kernel(
        _gather_body,
        mesh=mesh,
        compiler_params=pltpu.CompilerParams(needs_layout_passes=False),
        out_type=jax.ShapeDtypeStruct((NPAD, D), jnp.float32),
        scratch_types=(
            [pltpu.VMEM((NPAD // NW,), jnp.int32)]
            + [pltpu.VMEM((8, D), jnp.float32) for _ in range(_GNB)]
            + [pltpu.SemaphoreType.DMA for _ in range(2 * _GNB)]
        ),
    )
    return fn(inv_tok, xflat)


def _shared_body(x_ref, gw_ref, uw_ref, dw_ref, o_ref):
    x = x_ref[...].astype(jnp.bfloat16)
    gw = gw_ref[...].astype(jnp.bfloat16)
    uw = uw_ref[...].astype(jnp.bfloat16)
    g = jnp.dot(x, gw, preferred_element_type=jnp.float32)
    u = jnp.dot(x, uw, preferred_element_type=jnp.float32)
    h = (g * _sigmoid(g) * u).astype(jnp.bfloat16)
    o_ref[...] = jnp.dot(h, dw_ref[...].astype(jnp.bfloat16),
                         preferred_element_type=jnp.float32)


def _shared_ffn(xf, s_gate, s_up, s_down):
    nb = T // BLK
    return pl.pallas_call(
        _shared_body,
        grid=(nb,),
        in_specs=[
            pl.BlockSpec((BLK, D), lambda i: (i, 0)),
            pl.BlockSpec((D, I_S), lambda i: (0, 0)),
            pl.BlockSpec((D, I_S), lambda i: (0, 0)),
            pl.BlockSpec((I_S, D), lambda i: (0, 0)),
        ],
        out_specs=pl.BlockSpec((BLK, D), lambda i: (i, 0)),
        out_shape=jax.ShapeDtypeStruct((T, D), jnp.float32),
    )(xf, s_gate, s_up, s_down)


def _ffn_body(be_ref, xg_ref, gate_ref, gw_ref, uw_ref, dw_ref, o_ref):
    x = xg_ref[...].astype(jnp.bfloat16)
    g = jnp.dot(x, gw_ref[0].astype(jnp.bfloat16), preferred_element_type=jnp.float32)
    u = jnp.dot(x, uw_ref[0].astype(jnp.bfloat16), preferred_element_type=jnp.float32)
    h = (g * _sigmoid(g) * u).astype(jnp.bfloat16)
    y = jnp.dot(h, dw_ref[0].astype(jnp.bfloat16), preferred_element_type=jnp.float32)
    o_ref[...] = y * gate_ref[...]


def _expert_ffn(be, xg, inv_gate, e_gate, e_up, e_down):
    grid_spec = pltpu.PrefetchScalarGridSpec(
        num_scalar_prefetch=1,
        grid=(G,),
        in_specs=[
            pl.BlockSpec((BLK, D), lambda g, be_r: (g, 0)),
            pl.BlockSpec((BLK, 1), lambda g, be_r: (g, 0)),
            pl.BlockSpec((1, D, I_R), lambda g, be_r: (be_r[g], 0, 0)),
            pl.BlockSpec((1, D, I_R), lambda g, be_r: (be_r[g], 0, 0)),
            pl.BlockSpec((1, I_R, D), lambda g, be_r: (be_r[g], 0, 0)),
        ],
        out_specs=pl.BlockSpec((BLK, D), lambda g, be_r: (g, 0)),
    )
    return pl.pallas_call(
        _ffn_body,
        grid_spec=grid_spec,
        out_shape=jax.ShapeDtypeStruct((NPAD, D), jnp.float32),
    )(be, xg, inv_gate.reshape(NPAD, 1), e_gate, e_up, e_down)


def _combine_body(sh_h, yg_h, p1_h, p2_h, out_h, i1v, i2v, r1, r2, acc,
                  sem1, sem2):
    wid = lax.axis_index("s") * NC + lax.axis_index("c")
    per_w = T // NW
    chunk = 32
    base = wid * per_w
    for c in range(per_w // chunk):
        t0 = base + c * chunk
        pltpu.sync_copy(p1_h.at[pl.ds(t0, chunk)], i1v)
        pltpu.sync_copy(p2_h.at[pl.ds(t0, chunk)], i2v)
        cp1 = pltpu.async_copy(yg_h.at[i1v], r1, sem1)
        cp2 = pltpu.async_copy(yg_h.at[i2v], r2, sem2)
        pltpu.sync_copy(sh_h.at[pl.ds(t0, chunk)], acc)
        cp1.wait()
        cp2.wait()

        def row_body(i, carry):
            for j in range(D // 16):
                sl = pl.ds(j * 16, 16)
                acc[i, sl] = acc[i, sl] + r1[i, sl] + r2[i, sl]
            return carry
        lax.fori_loop(0, chunk, row_body, 0)
        pltpu.sync_copy(acc, out_h.at[pl.ds(t0, chunk)])


def _combine(shared, yg, p1, p2):
    mesh = plsc.VectorSubcoreMesh(core_axis_name="c", subcore_axis_name="s")
    fn = pl.kernel(
        _combine_body,
        mesh=mesh,
        compiler_params=pltpu.CompilerParams(needs_layout_passes=False),
        out_type=jax.ShapeDtypeStruct((T, D), jnp.float32),
        scratch_types=[
            pltpu.VMEM((32,), jnp.int32),
            pltpu.VMEM((32,), jnp.int32),
            pltpu.VMEM((32, D), jnp.float32),
            pltpu.VMEM((32, D), jnp.float32),
            pltpu.VMEM((32, D), jnp.float32),
            pltpu.SemaphoreType.DMA,
            pltpu.SemaphoreType.DMA,
        ],
    )
    return fn(shared, yg, p1, p2)


def kernel(x, s_gate, s_up, s_down, e_gate, e_up, e_down, router_w, expert_bias):
    B_, S_, D_ = x.shape
    xf = x.reshape(B_ * S_, D_)

    p1, p2, g1, g2, counts, loss, be = _router(xf, router_w, expert_bias)
    inv_tok, inv_gate = _build_inv(
        p1.reshape(T), p2.reshape(T), g1.reshape(T), g2.reshape(T))
    xg = _gather_rows(inv_tok, xf.reshape(T * D))
    shared = _shared_ffn(xf, s_gate, s_up, s_down)
    yg = _expert_ffn(be.reshape(32)[:G], xg, inv_gate, e_gate, e_up, e_down)
    out = _combine(shared, yg, p1.reshape(T), p2.reshape(T))

    output = out.reshape(B_, S_, D_)
    return (output, loss.reshape(()), counts.reshape(E))


# speedup vs baseline: 1.2276x; 1.2276x over previous
import functools

import jax
import jax.numpy as jnp
from jax import lax
from jax.experimental import pallas as pl
from jax.experimental.pallas import tpu as pltpu
from jax.experimental.pallas import tpu_sc as plsc

T = 2048
D = 1024
E = 8
K = 2
I_R = 512
I_S = 1024
ALPHA = 0.0001
BLK = 256
G = 24
NPAD = G * BLK
NEG = -3.0e38

NC, NS = 2, 16
NW = NC * NS


def _sigmoid(v):
    return 1.0 / (1.0 + jnp.exp(-v))


def _router_body(xf_ref, rw_ref, bias_ref, p1_ref, p2_ref, g1_ref, g2_ref,
                 cnt_ref, loss_ref, be_ref, xt_ref):
    xf = xf_ref[...]
    xt_ref[...] = xf.T
    logits = jnp.dot(xf, rw_ref[...], preferred_element_type=jnp.float32)
    aff = _sigmoid(logits)
    scores = aff + bias_ref[...]
    iota_e = lax.broadcasted_iota(jnp.int32, (T, E), 1)

    m1 = jnp.max(scores, axis=1, keepdims=True)
    i1 = jnp.min(jnp.where(scores == m1, iota_e, E), axis=1, keepdims=True)
    sel1 = iota_e == i1
    masked = jnp.where(sel1, NEG, scores)
    m2 = jnp.max(masked, axis=1, keepdims=True)
    i2 = jnp.min(jnp.where(masked == m2, iota_e, E), axis=1, keepdims=True)
    sel2 = iota_e == i2

    a1 = jnp.sum(jnp.where(sel1, aff, 0.0), axis=1, keepdims=True)
    a2 = jnp.sum(jnp.where(sel2, aff, 0.0), axis=1, keepdims=True)
    den = a1 + a2 + 1e-9
    g1_ref[...] = a1 / den
    g2_ref[...] = a2 / den

    mask = jnp.where(sel1 | sel2, 1.0, 0.0)
    counts_f = jnp.sum(mask, axis=0, keepdims=True)
    cnt_ref[...] = counts_f.astype(jnp.int32)

    rowsum = jnp.sum(aff, axis=1, keepdims=True) + 1e-9
    p_mean = jnp.sum(aff / rowsum, axis=0, keepdims=True) * (1.0 / T)
    f_bal = counts_f * (E / (K * T))
    loss_ref[...] = jnp.sum(f_bal * p_mean, axis=1, keepdims=True) * ALPHA

    r_i = lax.broadcasted_iota(jnp.int32, (BLK, BLK), 0)
    c_i = lax.broadcasted_iota(jnp.int32, (BLK, BLK), 1)
    tri = jnp.where(r_i >= c_i, 1.0, 0.0)
    nblk = T // BLK
    parts = []
    prefix = jnp.zeros((1, E), jnp.float32)
    for b in range(nblk):
        blk = lax.slice(mask, (b * BLK, 0), ((b + 1) * BLK, E))
        within = jnp.dot(tri, blk, preferred_element_type=jnp.float32)
        parts.append(within + prefix)
        prefix = prefix + lax.slice(within, (BLK - 1, 0), (BLK, E))
    rank = jnp.concatenate(parts, axis=0)

    nb_e = jnp.floor((counts_f + (BLK - 1)) * (1.0 / BLK))
    pc = nb_e * BLK
    s_r = lax.broadcasted_iota(jnp.int32, (E, E), 0)
    s_c = lax.broadcasted_iota(jnp.int32, (E, E), 1)
    strict = jnp.where(s_r < s_c, 1.0, 0.0)
    off = jnp.dot(pc, strict, preferred_element_type=jnp.float32)

    off_b = jnp.broadcast_to(off, (T, E))
    o1 = jnp.sum(jnp.where(sel1, off_b, 0.0), axis=1, keepdims=True)
    o2 = jnp.sum(jnp.where(sel2, off_b, 0.0), axis=1, keepdims=True)
    r1 = jnp.sum(jnp.where(sel1, rank, 0.0), axis=1, keepdims=True)
    r2 = jnp.sum(jnp.where(sel2, rank, 0.0), axis=1, keepdims=True)
    p1_ref[...] = (o1 + r1 - 1.0).astype(jnp.int32)
    p2_ref[...] = (o2 + r2 - 1.0).astype(jnp.int32)

    ends = off + pc
    g_f = lax.broadcasted_iota(jnp.int32, (1, 32), 1).astype(jnp.float32) * float(BLK)
    lane8 = lax.broadcasted_iota(jnp.int32, (1, E), 1)
    be = jnp.zeros((1, 32), jnp.float32)
    for e in range(E):
        end_e = jnp.sum(jnp.where(lane8 == e, ends, 0.0), axis=1, keepdims=True)
        be = be + jnp.where(end_e <= g_f, 1.0, 0.0)
    be_ref[...] = jnp.minimum(be, E - 1.0).astype(jnp.int32)


def _router(xf, router_w, expert_bias):
    return pl.pallas_call(
        _router_body,
        out_shape=(
            jax.ShapeDtypeStruct((T, 1), jnp.int32),
            jax.ShapeDtypeStruct((T, 1), jnp.int32),
            jax.ShapeDtypeStruct((T, 1), jnp.float32),
            jax.ShapeDtypeStruct((T, 1), jnp.float32),
            jax.ShapeDtypeStruct((1, E), jnp.int32),
            jax.ShapeDtypeStruct((1, 1), jnp.float32),
            jax.ShapeDtypeStruct((1, 32), jnp.int32),
            jax.ShapeDtypeStruct((D, T), jnp.float32),
        ),
    )(xf, router_w, expert_bias.reshape(1, E))


def _build_inv_body(p1_h, p2_h, g1_h, g2_h, itok_h, igate_h, it_v, ig_v, pv, gv):
    wid = lax.axis_index("s") * NC + lax.axis_index("c")

    @pl.when(wid == 0)
    def _():
        def zero_body(i, c):
            it_v[pl.ds(i * 16, 16)] = jnp.zeros((16,), jnp.int32)
            ig_v[pl.ds(i * 16, 16)] = jnp.zeros((16,), jnp.float32)
            return c
        lax.fori_loop(0, NPAD // 16, zero_body, 0)
        for p_h, g_h in ((p1_h, g1_h), (p2_h, g2_h)):
            pltpu.sync_copy(p_h, pv)
            pltpu.sync_copy(g_h, gv)

            def scat_body(i, c):
                sl = pl.ds(i * 16, 16)
                idx = pv[sl]
                toks = lax.iota(jnp.int32, 16) + i * 16
                plsc.store_scatter(it_v, [idx], toks)
                plsc.store_scatter(ig_v, [idx], gv[sl])
                return c
            lax.fori_loop(0, T // 16, scat_body, 0)
        pltpu.sync_copy(it_v, itok_h)
        pltpu.sync_copy(ig_v, igate_h)


def _build_inv(p1, p2, g1, g2):
    mesh = plsc.VectorSubcoreMesh(core_axis_name="c", subcore_axis_name="s")
    fn = pl.kernel(
        _build_inv_body,
        mesh=mesh,
        compiler_params=pltpu.CompilerParams(needs_layout_passes=False),
        out_type=(
            jax.ShapeDtypeStruct((NPAD,), jnp.int32),
            jax.ShapeDtypeStruct((NPAD,), jnp.float32),
        ),
        scratch_types=[
            pltpu.VMEM((NPAD,), jnp.int32),
            pltpu.VMEM((NPAD,), jnp.float32),
            pltpu.VMEM((T,), jnp.int32),
            pltpu.VMEM((T,), jnp.float32),
        ],
    )
    return fn(p1, p2, g1, g2)


_SCH = 512
_SNC = NPAD // _SCH
_CPW = D // NW


def _gather_body(itok_h, xt_h, xgt_h, xsl, idxb, ob0, ob1):
    wid = lax.axis_index("s") * NC + lax.axis_index("c")
    r0 = wid * _CPW
    pltpu.sync_copy(xt_h.at[pl.ds(r0, _CPW)], xsl)
    obufs = (ob0, ob1)
    for c in range(_SNC):
        pltpu.sync_copy(itok_h.at[pl.ds(c * _SCH, _SCH)], idxb)
        ob = obufs[c % 2]

        def vec_body(v, carry):
            tokv = idxb[pl.ds(v * 16, 16)]
            for j in range(_CPW):
                rows = jnp.full((16,), j, jnp.int32)
                ob[j, pl.ds(v * 16, 16)] = plsc.load_gather(xsl, [rows, tokv])
            return carry
        lax.fori_loop(0, _SCH // 16, vec_body, 0)
        pltpu.sync_copy(ob, xgt_h.at[pl.ds(r0, _CPW), pl.ds(c * _SCH, _SCH)])


def _gather_rows(inv_tok, xt):
    mesh = plsc.VectorSubcoreMesh(core_axis_name="c", subcore_axis_name="s")
    fn = pl.kernel(
        _gather_body,
        mesh=mesh,
        compiler_params=pltpu.CompilerParams(needs_layout_passes=False),
        out_type=jax.ShapeDtypeStruct((D, NPAD), jnp.float32),
        scratch_types=[
            pltpu.VMEM((_CPW, T), jnp.float32),
            pltpu.VMEM((_SCH,), jnp.int32),
            pltpu.VMEM((_CPW, _SCH), jnp.float32),
            pltpu.VMEM((_CPW, _SCH), jnp.float32),
        ],
    )
    return fn(inv_tok, xt)


def _shared_body(x_ref, gw_ref, uw_ref, dw_ref, o_ref):
    x = x_ref[...].astype(jnp.bfloat16)
    gw = gw_ref[...].astype(jnp.bfloat16)
    uw = uw_ref[...].astype(jnp.bfloat16)
    g = jnp.dot(x, gw, preferred_element_type=jnp.float32)
    u = jnp.dot(x, uw, preferred_element_type=jnp.float32)
    h = (g * _sigmoid(g) * u).astype(jnp.bfloat16)
    o_ref[...] = jnp.dot(h, dw_ref[...].astype(jnp.bfloat16),
                         preferred_element_type=jnp.float32)


def _shared_ffn(xf, s_gate, s_up, s_down):
    nb = T // BLK
    return pl.pallas_call(
        _shared_body,
        grid=(nb,),
        in_specs=[
            pl.BlockSpec((BLK, D), lambda i: (i, 0)),
            pl.BlockSpec((D, I_S), lambda i: (0, 0)),
            pl.BlockSpec((D, I_S), lambda i: (0, 0)),
            pl.BlockSpec((I_S, D), lambda i: (0, 0)),
        ],
        out_specs=pl.BlockSpec((BLK, D), lambda i: (i, 0)),
        out_shape=jax.ShapeDtypeStruct((T, D), jnp.float32),
    )(xf, s_gate, s_up, s_down)


def _ffn_body(be_ref, xg_ref, gate_ref, gw_ref, uw_ref, dw_ref, o_ref):
    xt = xg_ref[...].astype(jnp.bfloat16)
    dn = (((0,), (0,)), ((), ()))
    g = lax.dot_general(xt, gw_ref[0].astype(jnp.bfloat16), dn,
                        preferred_element_type=jnp.float32)
    u = lax.dot_general(xt, uw_ref[0].astype(jnp.bfloat16), dn,
                        preferred_element_type=jnp.float32)
    h = (g * _sigmoid(g) * u).astype(jnp.bfloat16)
    y = jnp.dot(h, dw_ref[0].astype(jnp.bfloat16), preferred_element_type=jnp.float32)
    o_ref[...] = y * gate_ref[...]


def _expert_ffn(be, xg, inv_gate, e_gate, e_up, e_down):
    grid_spec = pltpu.PrefetchScalarGridSpec(
        num_scalar_prefetch=1,
        grid=(G,),
        in_specs=[
            pl.BlockSpec((D, BLK), lambda g, be_r: (0, g)),
            pl.BlockSpec((BLK, 1), lambda g, be_r: (g, 0)),
            pl.BlockSpec((1, D, I_R), lambda g, be_r: (be_r[g], 0, 0)),
            pl.BlockSpec((1, D, I_R), lambda g, be_r: (be_r[g], 0, 0)),
            pl.BlockSpec((1, I_R, D), lambda g, be_r: (be_r[g], 0, 0)),
        ],
        out_specs=pl.BlockSpec((BLK, D), lambda g, be_r: (g, 0)),
    )
    return pl.pallas_call(
        _ffn_body,
        grid_spec=grid_spec,
        out_shape=jax.ShapeDtypeStruct((NPAD, D), jnp.float32),
    )(be, xg, inv_gate.reshape(NPAD, 1), e_gate, e_up, e_down)


def _combine_body(sh_h, yg_h, p1_h, p2_h, out_h, i1v, i2v, r1, r2, acc,
                  sem1, sem2):
    wid = lax.axis_index("s") * NC + lax.axis_index("c")
    per_w = T // NW
    chunk = 32
    base = wid * per_w
    for c in range(per_w // chunk):
        t0 = base + c * chunk
        pltpu.sync_copy(p1_h.at[pl.ds(t0, chunk)], i1v)
        pltpu.sync_copy(p2_h.at[pl.ds(t0, chunk)], i2v)
        cp1 = pltpu.async_copy(yg_h.at[i1v], r1, sem1)
        cp2 = pltpu.async_copy(yg_h.at[i2v], r2, sem2)
        pltpu.sync_copy(sh_h.at[pl.ds(t0, chunk)], acc)
        cp1.wait()
        cp2.wait()

        def row_body(i, carry):
            for j in range(D // 16):
                sl = pl.ds(j * 16, 16)
                acc[i, sl] = acc[i, sl] + r1[i, sl] + r2[i, sl]
            return carry
        lax.fori_loop(0, chunk, row_body, 0)
        pltpu.sync_copy(acc, out_h.at[pl.ds(t0, chunk)])


def _combine(shared, yg, p1, p2):
    mesh = plsc.VectorSubcoreMesh(core_axis_name="c", subcore_axis_name="s")
    fn = pl.kernel(
        _combine_body,
        mesh=mesh,
        compiler_params=pltpu.CompilerParams(needs_layout_passes=False),
        out_type=jax.ShapeDtypeStruct((T, D), jnp.float32),
        scratch_types=[
            pltpu.VMEM((32,), jnp.int32),
            pltpu.VMEM((32,), jnp.int32),
            pltpu.VMEM((32, D), jnp.float32),
            pltpu.VMEM((32, D), jnp.float32),
            pltpu.VMEM((32, D), jnp.float32),
            pltpu.SemaphoreType.DMA,
            pltpu.SemaphoreType.DMA,
        ],
    )
    return fn(shared, yg, p1, p2)


def kernel(x, s_gate, s_up, s_down, e_gate, e_up, e_down, router_w, expert_bias):
    B_, S_, D_ = x.shape
    xf = x.reshape(B_ * S_, D_)

    p1, p2, g1, g2, counts, loss, be, xt = _router(xf, router_w, expert_bias)
    inv_tok, inv_gate = _build_inv(
        p1.reshape(T), p2.reshape(T), g1.reshape(T), g2.reshape(T))
    xg = _gather_rows(inv_tok, xt)
    shared = _shared_ffn(xf, s_gate, s_up, s_down)
    yg = _expert_ffn(be.reshape(32)[:G], xg, inv_gate, e_gate, e_up, e_down)
    out = _combine(shared, yg, p1.reshape(T), p2.reshape(T))

    output = out.reshape(B_, S_, D_)
    return (output, loss.reshape(()), counts.reshape(E))


# speedup vs baseline: 1.2707x; 1.0352x over previous
import functools

import jax
import jax.numpy as jnp
from jax import lax
from jax.experimental import pallas as pl
from jax.experimental.pallas import tpu as pltpu
from jax.experimental.pallas import tpu_sc as plsc

T = 2048
D = 1024
E = 8
K = 2
I_R = 512
I_S = 1024
ALPHA = 0.0001
BLK = 256
G = 24
NPAD = G * BLK
NEG = -3.0e38

NC, NS = 2, 16
NW = NC * NS


def _sigmoid(v):
    return 1.0 / (1.0 + jnp.exp(-v))


def _router_body(xf_ref, rw_ref, bias_ref, p1_ref, p2_ref, g1_ref, g2_ref,
                 cnt_ref, loss_ref, be_ref, xt_ref):
    xf = xf_ref[...]
    xt_ref[...] = xf.T
    logits = jnp.dot(xf, rw_ref[...], preferred_element_type=jnp.float32)
    aff = _sigmoid(logits)
    scores = aff + bias_ref[...]
    iota_e = lax.broadcasted_iota(jnp.int32, (T, E), 1)

    m1 = jnp.max(scores, axis=1, keepdims=True)
    i1 = jnp.min(jnp.where(scores == m1, iota_e, E), axis=1, keepdims=True)
    sel1 = iota_e == i1
    masked = jnp.where(sel1, NEG, scores)
    m2 = jnp.max(masked, axis=1, keepdims=True)
    i2 = jnp.min(jnp.where(masked == m2, iota_e, E), axis=1, keepdims=True)
    sel2 = iota_e == i2

    a1 = jnp.sum(jnp.where(sel1, aff, 0.0), axis=1, keepdims=True)
    a2 = jnp.sum(jnp.where(sel2, aff, 0.0), axis=1, keepdims=True)
    den = a1 + a2 + 1e-9
    g1_ref[...] = (a1 / den).reshape(T)
    g2_ref[...] = (a2 / den).reshape(T)

    mask = jnp.where(sel1 | sel2, 1.0, 0.0)
    counts_f = jnp.sum(mask, axis=0, keepdims=True)
    cnt_ref[...] = counts_f.astype(jnp.int32)

    rowsum = jnp.sum(aff, axis=1, keepdims=True) + 1e-9
    p_mean = jnp.sum(aff / rowsum, axis=0, keepdims=True) * (1.0 / T)
    f_bal = counts_f * (E / (K * T))
    loss_ref[...] = jnp.sum(f_bal * p_mean, axis=1, keepdims=True) * ALPHA

    r_i = lax.broadcasted_iota(jnp.int32, (BLK, BLK), 0)
    c_i = lax.broadcasted_iota(jnp.int32, (BLK, BLK), 1)
    tri = jnp.where(r_i >= c_i, 1.0, 0.0)
    nblk = T // BLK
    parts = []
    prefix = jnp.zeros((1, E), jnp.float32)
    for b in range(nblk):
        blk = lax.slice(mask, (b * BLK, 0), ((b + 1) * BLK, E))
        within = jnp.dot(tri, blk, preferred_element_type=jnp.float32)
        parts.append(within + prefix)
        prefix = prefix + lax.slice(within, (BLK - 1, 0), (BLK, E))
    rank = jnp.concatenate(parts, axis=0)

    nb_e = jnp.floor((counts_f + (BLK - 1)) * (1.0 / BLK))
    pc = nb_e * BLK
    s_r = lax.broadcasted_iota(jnp.int32, (E, E), 0)
    s_c = lax.broadcasted_iota(jnp.int32, (E, E), 1)
    strict = jnp.where(s_r < s_c, 1.0, 0.0)
    off = jnp.dot(pc, strict, preferred_element_type=jnp.float32)

    off_b = jnp.broadcast_to(off, (T, E))
    o1 = jnp.sum(jnp.where(sel1, off_b, 0.0), axis=1, keepdims=True)
    o2 = jnp.sum(jnp.where(sel2, off_b, 0.0), axis=1, keepdims=True)
    r1 = jnp.sum(jnp.where(sel1, rank, 0.0), axis=1, keepdims=True)
    r2 = jnp.sum(jnp.where(sel2, rank, 0.0), axis=1, keepdims=True)
    p1_ref[...] = (o1 + r1 - 1.0).astype(jnp.int32).reshape(T)
    p2_ref[...] = (o2 + r2 - 1.0).astype(jnp.int32).reshape(T)

    ends = off + pc
    g_f = lax.broadcasted_iota(jnp.int32, (1, 32), 1).astype(jnp.float32) * float(BLK)
    lane8 = lax.broadcasted_iota(jnp.int32, (1, E), 1)
    be = jnp.zeros((1, 32), jnp.float32)
    for e in range(E):
        end_e = jnp.sum(jnp.where(lane8 == e, ends, 0.0), axis=1, keepdims=True)
        be = be + jnp.where(end_e <= g_f, 1.0, 0.0)
    be_ref[...] = jnp.minimum(be, E - 1.0).astype(jnp.int32)


def _router(xf, router_w, expert_bias):
    return pl.pallas_call(
        _router_body,
        out_shape=(
            jax.ShapeDtypeStruct((T,), jnp.int32),
            jax.ShapeDtypeStruct((T,), jnp.int32),
            jax.ShapeDtypeStruct((T,), jnp.float32),
            jax.ShapeDtypeStruct((T,), jnp.float32),
            jax.ShapeDtypeStruct((1, E), jnp.int32),
            jax.ShapeDtypeStruct((1, 1), jnp.float32),
            jax.ShapeDtypeStruct((1, 32), jnp.int32),
            jax.ShapeDtypeStruct((D, T), jnp.float32),
        ),
    )(xf, router_w, expert_bias.reshape(1, E))


def _build_inv_body(p1_h, p2_h, g1_h, g2_h, itok_h, igate_h, it_v, ig_v, pv, gv):
    wid = lax.axis_index("s") * NC + lax.axis_index("c")

    @pl.when(wid == 0)
    def _():
        def zero_body(i, c):
            it_v[pl.ds(i * 16, 16)] = jnp.zeros((16,), jnp.int32)
            ig_v[pl.ds(i * 16, 16)] = jnp.zeros((16,), jnp.float32)
            return c
        lax.fori_loop(0, NPAD // 16, zero_body, 0)
        for p_h, g_h in ((p1_h, g1_h), (p2_h, g2_h)):
            pltpu.sync_copy(p_h, pv)
            pltpu.sync_copy(g_h, gv)

            def scat_body(i, c):
                sl = pl.ds(i * 16, 16)
                idx = pv[sl]
                toks = lax.iota(jnp.int32, 16) + i * 16
                plsc.store_scatter(it_v, [idx], toks)
                plsc.store_scatter(ig_v, [idx], gv[sl])
                return c
            lax.fori_loop(0, T // 16, scat_body, 0)
        pltpu.sync_copy(it_v, itok_h)
        pltpu.sync_copy(ig_v, igate_h)


def _build_inv(p1, p2, g1, g2):
    mesh = plsc.VectorSubcoreMesh(core_axis_name="c", subcore_axis_name="s")
    fn = pl.kernel(
        _build_inv_body,
        mesh=mesh,
        compiler_params=pltpu.CompilerParams(needs_layout_passes=False),
        out_type=(
            jax.ShapeDtypeStruct((NPAD,), jnp.int32),
            jax.ShapeDtypeStruct((NPAD,), jnp.float32),
        ),
        scratch_types=[
            pltpu.VMEM((NPAD,), jnp.int32),
            pltpu.VMEM((NPAD,), jnp.float32),
            pltpu.VMEM((T,), jnp.int32),
            pltpu.VMEM((T,), jnp.float32),
        ],
    )
    return fn(p1, p2, g1, g2)


_SCH = 512
_SNC = NPAD // _SCH
_CPW = D // NW


def _gather_body(itok_h, xt_h, xgt_h, xsl, idxb, ob0, ob1):
    wid = lax.axis_index("s") * NC + lax.axis_index("c")
    r0 = wid * _CPW
    pltpu.sync_copy(xt_h.at[pl.ds(r0, _CPW)], xsl)
    obufs = (ob0, ob1)
    for c in range(_SNC):
        pltpu.sync_copy(itok_h.at[pl.ds(c * _SCH, _SCH)], idxb)
        ob = obufs[c % 2]

        def vec_body(v, carry):
            tokv = idxb[pl.ds(v * 16, 16)]
            for j in range(_CPW):
                rows = jnp.full((16,), j, jnp.int32)
                ob[j, pl.ds(v * 16, 16)] = plsc.load_gather(xsl, [rows, tokv])
            return carry
        lax.fori_loop(0, _SCH // 16, vec_body, 0)
        pltpu.sync_copy(ob, xgt_h.at[pl.ds(r0, _CPW), pl.ds(c * _SCH, _SCH)])


def _gather_rows(inv_tok, xt):
    mesh = plsc.VectorSubcoreMesh(core_axis_name="c", subcore_axis_name="s")
    fn = pl.kernel(
        _gather_body,
        mesh=mesh,
        compiler_params=pltpu.CompilerParams(needs_layout_passes=False),
        out_type=jax.ShapeDtypeStruct((D, NPAD), jnp.float32),
        scratch_types=[
            pltpu.VMEM((_CPW, T), jnp.float32),
            pltpu.VMEM((_SCH,), jnp.int32),
            pltpu.VMEM((_CPW, _SCH), jnp.float32),
            pltpu.VMEM((_CPW, _SCH), jnp.float32),
        ],
    )
    return fn(inv_tok, xt)


def _shared_body(x_ref, gw_ref, uw_ref, dw_ref, o_ref):
    x = x_ref[...].astype(jnp.bfloat16)
    g = jnp.dot(x, gw_ref[...], preferred_element_type=jnp.float32)
    u = jnp.dot(x, uw_ref[...], preferred_element_type=jnp.float32)
    h = (g * _sigmoid(g) * u).astype(jnp.bfloat16)
    o_ref[...] = jnp.dot(h, dw_ref[...], preferred_element_type=jnp.float32)


def _shared_ffn(xf, s_gate, s_up, s_down):
    nb = T // BLK
    return pl.pallas_call(
        _shared_body,
        grid=(nb,),
        in_specs=[
            pl.BlockSpec((BLK, D), lambda i: (i, 0)),
            pl.BlockSpec((D, I_S), lambda i: (0, 0)),
            pl.BlockSpec((D, I_S), lambda i: (0, 0)),
            pl.BlockSpec((I_S, D), lambda i: (0, 0)),
        ],
        out_specs=pl.BlockSpec((BLK, D), lambda i: (i, 0)),
        out_shape=jax.ShapeDtypeStruct((T, D), jnp.float32),
    )(xf, s_gate, s_up, s_down)


def _ffn_body(be_ref, xg_ref, gate_ref, gw_ref, uw_ref, dw_ref, o_ref):
    xt = xg_ref[...].astype(jnp.bfloat16)
    dn = (((0,), (0,)), ((), ()))
    g = lax.dot_general(xt, gw_ref[0], dn, preferred_element_type=jnp.float32)
    u = lax.dot_general(xt, uw_ref[0], dn, preferred_element_type=jnp.float32)
    h = (g * _sigmoid(g) * u).astype(jnp.bfloat16)
    y = jnp.dot(h, dw_ref[0], preferred_element_type=jnp.float32)
    o_ref[...] = y * gate_ref[...]


def _expert_ffn(be, xg, inv_gate, e_gate, e_up, e_down):
    grid_spec = pltpu.PrefetchScalarGridSpec(
        num_scalar_prefetch=1,
        grid=(G,),
        in_specs=[
            pl.BlockSpec((D, BLK), lambda g, be_r: (0, g)),
            pl.BlockSpec((BLK, 1), lambda g, be_r: (g, 0)),
            pl.BlockSpec((1, D, I_R), lambda g, be_r: (be_r[g], 0, 0)),
            pl.BlockSpec((1, D, I_R), lambda g, be_r: (be_r[g], 0, 0)),
            pl.BlockSpec((1, I_R, D), lambda g, be_r: (be_r[g], 0, 0)),
        ],
        out_specs=pl.BlockSpec((BLK, D), lambda g, be_r: (g, 0)),
    )
    return pl.pallas_call(
        _ffn_body,
        grid_spec=grid_spec,
        out_shape=jax.ShapeDtypeStruct((NPAD, D), jnp.float32),
    )(be, xg, inv_gate.reshape(NPAD, 1), e_gate, e_up, e_down)


def _combine_body(sh_h, yg_h, p1_h, p2_h, out_h, i1v, i2v, r1, r2, acc,
                  sem1, sem2):
    wid = lax.axis_index("s") * NC + lax.axis_index("c")
    per_w = T // NW
    chunk = 32
    base = wid * per_w
    for c in range(per_w // chunk):
        t0 = base + c * chunk
        pltpu.sync_copy(p1_h.at[pl.ds(t0, chunk)], i1v)
        pltpu.sync_copy(p2_h.at[pl.ds(t0, chunk)], i2v)
        cp1 = pltpu.async_copy(yg_h.at[i1v], r1, sem1)
        cp2 = pltpu.async_copy(yg_h.at[i2v], r2, sem2)
        pltpu.sync_copy(sh_h.at[pl.ds(t0, chunk)], acc)
        cp1.wait()
        cp2.wait()

        def row_body(i, carry):
            for j in range(D // 16):
                sl = pl.ds(j * 16, 16)
                acc[i, sl] = acc[i, sl] + r1[i, sl] + r2[i, sl]
            return carry
        lax.fori_loop(0, chunk, row_body, 0)
        pltpu.sync_copy(acc, out_h.at[pl.ds(t0, chunk)])


def _combine(shared, yg, p1, p2):
    mesh = plsc.VectorSubcoreMesh(core_axis_name="c", subcore_axis_name="s")
    fn = pl.kernel(
        _combine_body,
        mesh=mesh,
        compiler_params=pltpu.CompilerParams(needs_layout_passes=False),
        out_type=jax.ShapeDtypeStruct((T, D), jnp.float32),
        scratch_types=[
            pltpu.VMEM((32,), jnp.int32),
            pltpu.VMEM((32,), jnp.int32),
            pltpu.VMEM((32, D), jnp.float32),
            pltpu.VMEM((32, D), jnp.float32),
            pltpu.VMEM((32, D), jnp.float32),
            pltpu.SemaphoreType.DMA,
            pltpu.SemaphoreType.DMA,
        ],
    )
    return fn(shared, yg, p1, p2)


def kernel(x, s_gate, s_up, s_down, e_gate, e_up, e_down, router_w, expert_bias):
    B_, S_, D_ = x.shape
    xf = x.reshape(B_ * S_, D_)

    p1, p2, g1, g2, counts, loss, be, xt = _router(xf, router_w, expert_bias)
    inv_tok, inv_gate = _build_inv(p1, p2, g1, g2)
    xg = _gather_rows(inv_tok, xt)
    shared = _shared_ffn(xf, s_gate.astype(jnp.bfloat16),
                         s_up.astype(jnp.bfloat16), s_down.astype(jnp.bfloat16))
    yg = _expert_ffn(be.reshape(32)[:G], xg, inv_gate,
                     e_gate.astype(jnp.bfloat16), e_up.astype(jnp.bfloat16),
                     e_down.astype(jnp.bfloat16))
    out = _combine(shared, yg, p1, p2)

    output = out.reshape(B_, S_, D_)
    return (output, loss.reshape(()), counts.reshape(E))


# speedup vs baseline: 1.3159x; 1.0355x over previous
import functools

import jax
import jax.numpy as jnp
from jax import lax
from jax.experimental import pallas as pl
from jax.experimental.pallas import tpu as pltpu
from jax.experimental.pallas import tpu_sc as plsc

T = 2048
D = 1024
E = 8
K = 2
I_R = 512
I_S = 1024
ALPHA = 0.0001
BLK = 256
G = 24
NPAD = G * BLK
NEG = -3.0e38

NC, NS = 2, 16
NW = NC * NS


def _sigmoid(v):
    return 1.0 / (1.0 + jnp.exp(-v))


def _router_body(xf_ref, rw_ref, bias_ref, p1_ref, p2_ref, g1_ref, g2_ref,
                 cnt_ref, loss_ref, be_ref, xt_ref, used_ref):
    xf = xf_ref[...]
    xt_ref[...] = xf.T
    logits = jnp.dot(xf, rw_ref[...], preferred_element_type=jnp.float32)
    aff = _sigmoid(logits)
    scores = aff + bias_ref[...]
    iota_e = lax.broadcasted_iota(jnp.int32, (T, E), 1)

    m1 = jnp.max(scores, axis=1, keepdims=True)
    i1 = jnp.min(jnp.where(scores == m1, iota_e, E), axis=1, keepdims=True)
    sel1 = iota_e == i1
    masked = jnp.where(sel1, NEG, scores)
    m2 = jnp.max(masked, axis=1, keepdims=True)
    i2 = jnp.min(jnp.where(masked == m2, iota_e, E), axis=1, keepdims=True)
    sel2 = iota_e == i2

    a1 = jnp.sum(jnp.where(sel1, aff, 0.0), axis=1, keepdims=True)
    a2 = jnp.sum(jnp.where(sel2, aff, 0.0), axis=1, keepdims=True)
    den = a1 + a2 + 1e-9
    g1_ref[...] = (a1 / den).reshape(T)
    g2_ref[...] = (a2 / den).reshape(T)

    mask = jnp.where(sel1 | sel2, 1.0, 0.0)
    counts_f = jnp.sum(mask, axis=0, keepdims=True)
    cnt_ref[...] = counts_f.astype(jnp.int32)

    rowsum = jnp.sum(aff, axis=1, keepdims=True) + 1e-9
    p_mean = jnp.sum(aff / rowsum, axis=0, keepdims=True) * (1.0 / T)
    f_bal = counts_f * (E / (K * T))
    loss_ref[...] = jnp.sum(f_bal * p_mean, axis=1, keepdims=True) * ALPHA

    r_i = lax.broadcasted_iota(jnp.int32, (BLK, BLK), 0)
    c_i = lax.broadcasted_iota(jnp.int32, (BLK, BLK), 1)
    tri = jnp.where(r_i >= c_i, 1.0, 0.0)
    nblk = T // BLK
    parts = []
    prefix = jnp.zeros((1, E), jnp.float32)
    for b in range(nblk):
        blk = lax.slice(mask, (b * BLK, 0), ((b + 1) * BLK, E))
        within = jnp.dot(tri, blk, preferred_element_type=jnp.float32)
        parts.append(within + prefix)
        prefix = prefix + lax.slice(within, (BLK - 1, 0), (BLK, E))
    rank = jnp.concatenate(parts, axis=0)

    nb_e = jnp.floor((counts_f + (BLK - 1)) * (1.0 / BLK))
    pc = nb_e * BLK
    s_r = lax.broadcasted_iota(jnp.int32, (E, E), 0)
    s_c = lax.broadcasted_iota(jnp.int32, (E, E), 1)
    strict = jnp.where(s_r < s_c, 1.0, 0.0)
    off = jnp.dot(pc, strict, preferred_element_type=jnp.float32)

    off_b = jnp.broadcast_to(off, (T, E))
    o1 = jnp.sum(jnp.where(sel1, off_b, 0.0), axis=1, keepdims=True)
    o2 = jnp.sum(jnp.where(sel2, off_b, 0.0), axis=1, keepdims=True)
    r1 = jnp.sum(jnp.where(sel1, rank, 0.0), axis=1, keepdims=True)
    r2 = jnp.sum(jnp.where(sel2, rank, 0.0), axis=1, keepdims=True)
    p1_ref[...] = (o1 + r1 - 1.0).astype(jnp.int32).reshape(T)
    p2_ref[...] = (o2 + r2 - 1.0).astype(jnp.int32).reshape(T)

    ends = off + pc
    g_f = lax.broadcasted_iota(jnp.int32, (1, 32), 1).astype(jnp.float32) * float(BLK)
    lane8 = lax.broadcasted_iota(jnp.int32, (1, E), 1)
    be = jnp.zeros((1, 32), jnp.float32)
    for e in range(E):
        end_e = jnp.sum(jnp.where(lane8 == e, ends, 0.0), axis=1, keepdims=True)
        be = be + jnp.where(end_e <= g_f, 1.0, 0.0)
    be_ref[...] = jnp.minimum(be, E - 1.0).astype(jnp.int32)
    used_ref[...] = jnp.broadcast_to(jnp.sum(pc, axis=1, keepdims=True),
                                     (1, 16)).astype(jnp.int32)


def _router(xf, router_w, expert_bias):
    return pl.pallas_call(
        _router_body,
        out_shape=(
            jax.ShapeDtypeStruct((T,), jnp.int32),
            jax.ShapeDtypeStruct((T,), jnp.int32),
            jax.ShapeDtypeStruct((T,), jnp.float32),
            jax.ShapeDtypeStruct((T,), jnp.float32),
            jax.ShapeDtypeStruct((1, E), jnp.int32),
            jax.ShapeDtypeStruct((1, 1), jnp.float32),
            jax.ShapeDtypeStruct((1, 32), jnp.int32),
            jax.ShapeDtypeStruct((D, T), jnp.float32),
            jax.ShapeDtypeStruct((1, 16), jnp.int32),
        ),
    )(xf, router_w, expert_bias.reshape(1, E))


def _build_inv_body(p1_h, p2_h, g1_h, g2_h, itok_h, igate_h, it_v, ig_v, pv, gv):
    wid = lax.axis_index("s") * NC + lax.axis_index("c")

    @pl.when(wid == 0)
    def _():
        def zero_body(i, c):
            it_v[pl.ds(i * 16, 16)] = jnp.zeros((16,), jnp.int32)
            ig_v[pl.ds(i * 16, 16)] = jnp.zeros((16,), jnp.float32)
            return c
        lax.fori_loop(0, NPAD // 16, zero_body, 0)
        for p_h, g_h in ((p1_h, g1_h), (p2_h, g2_h)):
            pltpu.sync_copy(p_h, pv)
            pltpu.sync_copy(g_h, gv)

            def scat_body(i, c):
                sl = pl.ds(i * 16, 16)
                idx = pv[sl]
                toks = lax.iota(jnp.int32, 16) + i * 16
                plsc.store_scatter(it_v, [idx], toks)
                plsc.store_scatter(ig_v, [idx], gv[sl])
                return c
            lax.fori_loop(0, T // 16, scat_body, 0)
        pltpu.sync_copy(it_v, itok_h)
        pltpu.sync_copy(ig_v, igate_h)


def _build_inv(p1, p2, g1, g2):
    mesh = plsc.VectorSubcoreMesh(core_axis_name="c", subcore_axis_name="s")
    fn = pl.kernel(
        _build_inv_body,
        mesh=mesh,
        compiler_params=pltpu.CompilerParams(needs_layout_passes=False),
        out_type=(
            jax.ShapeDtypeStruct((NPAD,), jnp.int32),
            jax.ShapeDtypeStruct((NPAD,), jnp.float32),
        ),
        scratch_types=[
            pltpu.VMEM((NPAD,), jnp.int32),
            pltpu.VMEM((NPAD,), jnp.float32),
            pltpu.VMEM((T,), jnp.int32),
            pltpu.VMEM((T,), jnp.float32),
        ],
    )
    return fn(p1, p2, g1, g2)


_SCH = 512
_SNC = NPAD // _SCH
_CPW = D // NW


def _gather_body(itok_h, used_h, xt_h, xgt_h, xsl, idxb, uv, ob0, ob1):
    wid = lax.axis_index("s") * NC + lax.axis_index("c")
    r0 = wid * _CPW
    pltpu.sync_copy(used_h, uv)
    pltpu.sync_copy(xt_h.at[pl.ds(r0, _CPW)], xsl)
    used = jnp.max(uv[0, pl.ds(0, 16)])
    obufs = (ob0, ob1)
    for c in range(_SNC):
        @pl.when(used > c * _SCH)
        def _():
            pltpu.sync_copy(itok_h.at[pl.ds(c * _SCH, _SCH)], idxb)
            ob = obufs[c % 2]

            def vec_body(v, carry):
                tokv = idxb[pl.ds(v * 16, 16)]
                for j in range(_CPW):
                    rows = jnp.full((16,), j, jnp.int32)
                    ob[j, pl.ds(v * 16, 16)] = plsc.load_gather(xsl, [rows, tokv])
                return carry
            lax.fori_loop(0, _SCH // 16, vec_body, 0)
            pltpu.sync_copy(ob, xgt_h.at[pl.ds(r0, _CPW), pl.ds(c * _SCH, _SCH)])


def _gather_rows(inv_tok, used, xt):
    mesh = plsc.VectorSubcoreMesh(core_axis_name="c", subcore_axis_name="s")
    fn = pl.kernel(
        _gather_body,
        mesh=mesh,
        compiler_params=pltpu.CompilerParams(needs_layout_passes=False),
        out_type=jax.ShapeDtypeStruct((D, NPAD), jnp.float32),
        scratch_types=[
            pltpu.VMEM((_CPW, T), jnp.float32),
            pltpu.VMEM((_SCH,), jnp.int32),
            pltpu.VMEM((1, 16), jnp.int32),
            pltpu.VMEM((_CPW, _SCH), jnp.float32),
            pltpu.VMEM((_CPW, _SCH), jnp.float32),
        ],
    )
    return fn(inv_tok, used, xt)


def _shared_body(x_ref, gw_ref, uw_ref, dw_ref, o_ref):
    x = x_ref[...].astype(jnp.bfloat16)
    g = jnp.dot(x, gw_ref[...], preferred_element_type=jnp.float32)
    u = jnp.dot(x, uw_ref[...], preferred_element_type=jnp.float32)
    h = (g * _sigmoid(g) * u).astype(jnp.bfloat16)
    o_ref[...] = jnp.dot(h, dw_ref[...], preferred_element_type=jnp.float32)


def _shared_ffn(xf, s_gate, s_up, s_down):
    nb = T // BLK
    return pl.pallas_call(
        _shared_body,
        grid=(nb,),
        in_specs=[
            pl.BlockSpec((BLK, D), lambda i: (i, 0)),
            pl.BlockSpec((D, I_S), lambda i: (0, 0)),
            pl.BlockSpec((D, I_S), lambda i: (0, 0)),
            pl.BlockSpec((I_S, D), lambda i: (0, 0)),
        ],
        out_specs=pl.BlockSpec((BLK, D), lambda i: (i, 0)),
        out_shape=jax.ShapeDtypeStruct((T, D), jnp.float32),
    )(xf, s_gate, s_up, s_down)


def _ffn_body(be_ref, xg_ref, gate_ref, gw_ref, uw_ref, dw_ref, o_ref):
    xt = xg_ref[...].astype(jnp.bfloat16)
    dn = (((0,), (0,)), ((), ()))
    g = lax.dot_general(xt, gw_ref[0], dn, preferred_element_type=jnp.float32)
    u = lax.dot_general(xt, uw_ref[0], dn, preferred_element_type=jnp.float32)
    h = (g * _sigmoid(g) * u).astype(jnp.bfloat16)
    y = jnp.dot(h, dw_ref[0], preferred_element_type=jnp.float32)
    o_ref[...] = y * gate_ref[...]


def _expert_ffn(be, xg, inv_gate, e_gate, e_up, e_down):
    grid_spec = pltpu.PrefetchScalarGridSpec(
        num_scalar_prefetch=1,
        grid=(G,),
        in_specs=[
            pl.BlockSpec((D, BLK), lambda g, be_r: (0, g)),
            pl.BlockSpec((BLK, 1), lambda g, be_r: (g, 0)),
            pl.BlockSpec((1, D, I_R), lambda g, be_r: (be_r[g], 0, 0)),
            pl.BlockSpec((1, D, I_R), lambda g, be_r: (be_r[g], 0, 0)),
            pl.BlockSpec((1, I_R, D), lambda g, be_r: (be_r[g], 0, 0)),
        ],
        out_specs=pl.BlockSpec((BLK, D), lambda g, be_r: (g, 0)),
    )
    return pl.pallas_call(
        _ffn_body,
        grid_spec=grid_spec,
        out_shape=jax.ShapeDtypeStruct((NPAD, D), jnp.float32),
    )(be, xg, inv_gate.reshape(NPAD, 1), e_gate, e_up, e_down)


def _combine_body(sh_h, yg_h, p1_h, p2_h, out_h, i1v, i2v, r1, r2, acc,
                  sem1, sem2):
    wid = lax.axis_index("s") * NC + lax.axis_index("c")
    per_w = T // NW
    chunk = 32
    base = wid * per_w
    for c in range(per_w // chunk):
        t0 = base + c * chunk
        pltpu.sync_copy(p1_h.at[pl.ds(t0, chunk)], i1v)
        pltpu.sync_copy(p2_h.at[pl.ds(t0, chunk)], i2v)
        cp1 = pltpu.async_copy(yg_h.at[i1v], r1, sem1)
        cp2 = pltpu.async_copy(yg_h.at[i2v], r2, sem2)
        pltpu.sync_copy(sh_h.at[pl.ds(t0, chunk)], acc)
        cp1.wait()
        cp2.wait()

        def row_body(i, carry):
            for j in range(D // 16):
                sl = pl.ds(j * 16, 16)
                acc[i, sl] = acc[i, sl] + r1[i, sl] + r2[i, sl]
            return carry
        lax.fori_loop(0, chunk, row_body, 0)
        pltpu.sync_copy(acc, out_h.at[pl.ds(t0, chunk)])


def _combine(shared, yg, p1, p2):
    mesh = plsc.VectorSubcoreMesh(core_axis_name="c", subcore_axis_name="s")
    fn = pl.kernel(
        _combine_body,
        mesh=mesh,
        compiler_params=pltpu.CompilerParams(needs_layout_passes=False),
        out_type=jax.ShapeDtypeStruct((T, D), jnp.float32),
        scratch_types=[
            pltpu.VMEM((32,), jnp.int32),
            pltpu.VMEM((32,), jnp.int32),
            pltpu.VMEM((32, D), jnp.float32),
            pltpu.VMEM((32, D), jnp.float32),
            pltpu.VMEM((32, D), jnp.float32),
            pltpu.SemaphoreType.DMA,
            pltpu.SemaphoreType.DMA,
        ],
    )
    return fn(shared, yg, p1, p2)


def kernel(x, s_gate, s_up, s_down, e_gate, e_up, e_down, router_w, expert_bias):
    B_, S_, D_ = x.shape
    xf = x.reshape(B_ * S_, D_)

    p1, p2, g1, g2, counts, loss, be, xt, used = _router(xf, router_w, expert_bias)
    inv_tok, inv_gate = _build_inv(p1, p2, g1, g2)
    xg = _gather_rows(inv_tok, used, xt)
    shared = _shared_ffn(xf, s_gate.astype(jnp.bfloat16),
                         s_up.astype(jnp.bfloat16), s_down.astype(jnp.bfloat16))
    yg = _expert_ffn(be.reshape(32)[:G], xg, inv_gate,
                     e_gate.astype(jnp.bfloat16), e_up.astype(jnp.bfloat16),
                     e_down.astype(jnp.bfloat16))
    out = _combine(shared, yg, p1, p2)

    output = out.reshape(B_, S_, D_)
    return (output, loss.reshape(()), counts.reshape(E))


# speedup vs baseline: 1.3483x; 1.0246x over previous
import functools

import jax
import jax.numpy as jnp
from jax import lax
from jax.experimental import pallas as pl
from jax.experimental.pallas import tpu as pltpu
from jax.experimental.pallas import tpu_sc as plsc

T = 2048
D = 1024
E = 8
K = 2
I_R = 512
I_S = 1024
ALPHA = 0.0001
BLK = 256
G = 24
NPAD = G * BLK
NEG = -3.0e38

NC, NS = 2, 16
NW = NC * NS


def _sigmoid(v):
    return 1.0 / (1.0 + jnp.exp(-v))


def _router_body(xf_ref, rw_ref, bias_ref, p1_ref, p2_ref, g1_ref, g2_ref,
                 cnt_ref, loss_ref, be_ref, xt_ref, used_ref):
    xf = xf_ref[...]
    xt_ref[...] = xf.T
    logits = jnp.dot(xf, rw_ref[...], preferred_element_type=jnp.float32)
    aff = _sigmoid(logits)
    scores = aff + bias_ref[...]
    iota_e = lax.broadcasted_iota(jnp.int32, (T, E), 1)

    m1 = jnp.max(scores, axis=1, keepdims=True)
    i1 = jnp.min(jnp.where(scores == m1, iota_e, E), axis=1, keepdims=True)
    sel1 = iota_e == i1
    masked = jnp.where(sel1, NEG, scores)
    m2 = jnp.max(masked, axis=1, keepdims=True)
    i2 = jnp.min(jnp.where(masked == m2, iota_e, E), axis=1, keepdims=True)
    sel2 = iota_e == i2

    a1 = jnp.sum(jnp.where(sel1, aff, 0.0), axis=1, keepdims=True)
    a2 = jnp.sum(jnp.where(sel2, aff, 0.0), axis=1, keepdims=True)
    den = a1 + a2 + 1e-9
    g1_ref[...] = (a1 / den).reshape(T)
    g2_ref[...] = (a2 / den).reshape(T)

    mask = jnp.where(sel1 | sel2, 1.0, 0.0)
    counts_f = jnp.sum(mask, axis=0, keepdims=True)
    cnt_ref[...] = counts_f.astype(jnp.int32)

    rowsum = jnp.sum(aff, axis=1, keepdims=True) + 1e-9
    p_mean = jnp.sum(aff / rowsum, axis=0, keepdims=True) * (1.0 / T)
    f_bal = counts_f * (E / (K * T))
    loss_ref[...] = jnp.sum(f_bal * p_mean, axis=1, keepdims=True) * ALPHA

    r_i = lax.broadcasted_iota(jnp.int32, (BLK, BLK), 0)
    c_i = lax.broadcasted_iota(jnp.int32, (BLK, BLK), 1)
    tri = jnp.where(r_i >= c_i, 1.0, 0.0)
    nblk = T // BLK
    parts = []
    prefix = jnp.zeros((1, E), jnp.float32)
    for b in range(nblk):
        blk = lax.slice(mask, (b * BLK, 0), ((b + 1) * BLK, E))
        within = jnp.dot(tri, blk, preferred_element_type=jnp.float32)
        parts.append(within + prefix)
        prefix = prefix + lax.slice(within, (BLK - 1, 0), (BLK, E))
    rank = jnp.concatenate(parts, axis=0)

    nb_e = jnp.floor((counts_f + (BLK - 1)) * (1.0 / BLK))
    pc = nb_e * BLK
    s_r = lax.broadcasted_iota(jnp.int32, (E, E), 0)
    s_c = lax.broadcasted_iota(jnp.int32, (E, E), 1)
    strict = jnp.where(s_r < s_c, 1.0, 0.0)
    off = jnp.dot(pc, strict, preferred_element_type=jnp.float32)

    off_b = jnp.broadcast_to(off, (T, E))
    o1 = jnp.sum(jnp.where(sel1, off_b, 0.0), axis=1, keepdims=True)
    o2 = jnp.sum(jnp.where(sel2, off_b, 0.0), axis=1, keepdims=True)
    r1 = jnp.sum(jnp.where(sel1, rank, 0.0), axis=1, keepdims=True)
    r2 = jnp.sum(jnp.where(sel2, rank, 0.0), axis=1, keepdims=True)
    p1_ref[...] = (o1 + r1 - 1.0).astype(jnp.int32).reshape(T)
    p2_ref[...] = (o2 + r2 - 1.0).astype(jnp.int32).reshape(T)

    ends = off + pc
    g_f = lax.broadcasted_iota(jnp.int32, (1, 32), 1).astype(jnp.float32) * float(BLK)
    lane8 = lax.broadcasted_iota(jnp.int32, (1, E), 1)
    be = jnp.zeros((1, 32), jnp.float32)
    for e in range(E):
        end_e = jnp.sum(jnp.where(lane8 == e, ends, 0.0), axis=1, keepdims=True)
        be = be + jnp.where(end_e <= g_f, 1.0, 0.0)
    be_ref[...] = jnp.minimum(be, E - 1.0).astype(jnp.int32)
    used_ref[...] = jnp.broadcast_to(jnp.sum(pc, axis=1, keepdims=True),
                                     (1, 16)).astype(jnp.int32)


def _router(xf, router_w, expert_bias):
    return pl.pallas_call(
        _router_body,
        out_shape=(
            jax.ShapeDtypeStruct((T,), jnp.int32),
            jax.ShapeDtypeStruct((T,), jnp.int32),
            jax.ShapeDtypeStruct((T,), jnp.float32),
            jax.ShapeDtypeStruct((T,), jnp.float32),
            jax.ShapeDtypeStruct((1, E), jnp.int32),
            jax.ShapeDtypeStruct((1, 1), jnp.float32),
            jax.ShapeDtypeStruct((1, 32), jnp.int32),
            jax.ShapeDtypeStruct((D, T), jnp.float32),
            jax.ShapeDtypeStruct((1, 16), jnp.int32),
        ),
    )(xf, router_w, expert_bias.reshape(1, E))


def _build_inv_body(p1_h, p2_h, g1_h, g2_h, itok_h, igate_h, it_v, ig_v, pv, gv):
    wid = lax.axis_index("s") * NC + lax.axis_index("c")

    @pl.when(wid == 0)
    def _():
        def zero_body(i, c):
            it_v[pl.ds(i * 16, 16)] = jnp.zeros((16,), jnp.int32)
            ig_v[pl.ds(i * 16, 16)] = jnp.zeros((16,), jnp.float32)
            return c
        lax.fori_loop(0, NPAD // 16, zero_body, 0)
        for p_h, g_h in ((p1_h, g1_h), (p2_h, g2_h)):
            pltpu.sync_copy(p_h, pv)
            pltpu.sync_copy(g_h, gv)

            def scat_body(i, c):
                sl = pl.ds(i * 16, 16)
                idx = pv[sl]
                toks = lax.iota(jnp.int32, 16) + i * 16
                plsc.store_scatter(it_v, [idx], toks)
                plsc.store_scatter(ig_v, [idx], gv[sl])
                return c
            lax.fori_loop(0, T // 16, scat_body, 0)
        pltpu.sync_copy(it_v, itok_h)
        pltpu.sync_copy(ig_v, igate_h)


def _build_inv(p1, p2, g1, g2):
    mesh = plsc.VectorSubcoreMesh(core_axis_name="c", subcore_axis_name="s")
    fn = pl.kernel(
        _build_inv_body,
        mesh=mesh,
        compiler_params=pltpu.CompilerParams(needs_layout_passes=False),
        out_type=(
            jax.ShapeDtypeStruct((NPAD,), jnp.int32),
            jax.ShapeDtypeStruct((NPAD,), jnp.float32),
        ),
        scratch_types=[
            pltpu.VMEM((NPAD,), jnp.int32),
            pltpu.VMEM((NPAD,), jnp.float32),
            pltpu.VMEM((T,), jnp.int32),
            pltpu.VMEM((T,), jnp.float32),
        ],
    )
    return fn(p1, p2, g1, g2)


_SCH = 512
_SNC = NPAD // _SCH
_CPW = D // NW


def _gather_body(itok_h, used_h, xt_h, xgt_h, xsl, idxb, uv, ob0, ob1):
    wid = lax.axis_index("s") * NC + lax.axis_index("c")
    r0 = wid * _CPW
    pltpu.sync_copy(used_h, uv)
    pltpu.sync_copy(xt_h.at[pl.ds(r0, _CPW)], xsl)
    used = jnp.max(uv[0, pl.ds(0, 16)])
    obufs = (ob0, ob1)
    for c in range(_SNC):
        @pl.when(used > c * _SCH)
        def _():
            pltpu.sync_copy(itok_h.at[pl.ds(c * _SCH, _SCH)], idxb)
            ob = obufs[c % 2]

            def vec_body(v, carry):
                tokv = idxb[pl.ds(v * 16, 16)]
                for j in range(_CPW):
                    rows = jnp.full((16,), j, jnp.int32)
                    ob[j, pl.ds(v * 16, 16)] = plsc.load_gather(xsl, [rows, tokv])
                return carry
            lax.fori_loop(0, _SCH // 16, vec_body, 0)
            pltpu.sync_copy(ob, xgt_h.at[pl.ds(r0, _CPW), pl.ds(c * _SCH, _SCH)])


def _gather_rows(inv_tok, used, xt):
    mesh = plsc.VectorSubcoreMesh(core_axis_name="c", subcore_axis_name="s")
    fn = pl.kernel(
        _gather_body,
        mesh=mesh,
        compiler_params=pltpu.CompilerParams(needs_layout_passes=False),
        out_type=jax.ShapeDtypeStruct((D, NPAD), jnp.float32),
        scratch_types=[
            pltpu.VMEM((_CPW, T), jnp.float32),
            pltpu.VMEM((_SCH,), jnp.int32),
            pltpu.VMEM((1, 16), jnp.int32),
            pltpu.VMEM((_CPW, _SCH), jnp.float32),
            pltpu.VMEM((_CPW, _SCH), jnp.float32),
        ],
    )
    return fn(inv_tok, used, xt)


def _shared_body(x_ref, gw_ref, uw_ref, dw_ref, o_ref):
    x = x_ref[...].astype(jnp.bfloat16)
    g = jnp.dot(x, gw_ref[...], preferred_element_type=jnp.float32)
    u = jnp.dot(x, uw_ref[...], preferred_element_type=jnp.float32)
    h = (g * _sigmoid(g) * u).astype(jnp.bfloat16)
    o_ref[...] = jnp.dot(h, dw_ref[...], preferred_element_type=jnp.float32)


def _shared_ffn(xf, s_gate, s_up, s_down):
    nb = T // BLK
    return pl.pallas_call(
        _shared_body,
        grid=(nb,),
        in_specs=[
            pl.BlockSpec((BLK, D), lambda i: (i, 0)),
            pl.BlockSpec((D, I_S), lambda i: (0, 0)),
            pl.BlockSpec((D, I_S), lambda i: (0, 0)),
            pl.BlockSpec((I_S, D), lambda i: (0, 0)),
        ],
        out_specs=pl.BlockSpec((BLK, D), lambda i: (i, 0)),
        out_shape=jax.ShapeDtypeStruct((T, D), jnp.float32),
    )(xf, s_gate, s_up, s_down)


def _ffn_body(be_ref, xg_ref, gate_ref, gw_ref, uw_ref, dw_ref, o_ref):
    xt = xg_ref[...].astype(jnp.bfloat16)
    dn = (((0,), (0,)), ((), ()))
    g = lax.dot_general(xt, gw_ref[0], dn, preferred_element_type=jnp.float32)
    u = lax.dot_general(xt, uw_ref[0], dn, preferred_element_type=jnp.float32)
    h = (g * _sigmoid(g) * u).astype(jnp.bfloat16)
    y = jnp.dot(h, dw_ref[0], preferred_element_type=jnp.float32)
    o_ref[...] = y * gate_ref[...]


def _expert_ffn(be, xg, inv_gate, e_gate, e_up, e_down):
    grid_spec = pltpu.PrefetchScalarGridSpec(
        num_scalar_prefetch=1,
        grid=(G,),
        in_specs=[
            pl.BlockSpec((D, BLK), lambda g, be_r: (0, g)),
            pl.BlockSpec((BLK, 1), lambda g, be_r: (g, 0)),
            pl.BlockSpec((1, D, I_R), lambda g, be_r: (be_r[g], 0, 0)),
            pl.BlockSpec((1, D, I_R), lambda g, be_r: (be_r[g], 0, 0)),
            pl.BlockSpec((1, I_R, D), lambda g, be_r: (be_r[g], 0, 0)),
        ],
        out_specs=pl.BlockSpec((BLK, D), lambda g, be_r: (g, 0)),
    )
    return pl.pallas_call(
        _ffn_body,
        grid_spec=grid_spec,
        out_shape=jax.ShapeDtypeStruct((NPAD, D), jnp.float32),
    )(be, xg, inv_gate.reshape(NPAD, 1), e_gate, e_up, e_down)


def _combine_body(sh_h, yg_h, p1_h, p2_h, out_h, *rest):
    i1v = rest[0:2]
    i2v = rest[2:4]
    r1 = rest[4:6]
    r2 = rest[6:8]
    acc = rest[8:10]
    s1 = rest[10:12]
    s2 = rest[12:14]
    ss = rest[14:16]
    ws = rest[16:18]
    wid = lax.axis_index("s") * NC + lax.axis_index("c")
    per_w = T // NW
    chunk = 16
    nchk = per_w // chunk
    base = wid * per_w
    cps = [None] * nchk
    wcp = [None] * nchk

    def start(c):
        b = c % 2
        t0 = base + c * chunk
        pltpu.sync_copy(p1_h.at[pl.ds(t0, chunk)], i1v[b])
        pltpu.sync_copy(p2_h.at[pl.ds(t0, chunk)], i2v[b])
        cps[c] = (pltpu.async_copy(yg_h.at[i1v[b]], r1[b], s1[b]),
                  pltpu.async_copy(yg_h.at[i2v[b]], r2[b], s2[b]),
                  pltpu.async_copy(sh_h.at[pl.ds(t0, chunk)], acc[b], ss[b]))

    start(0)
    start(1)
    for c in range(nchk):
        b = c % 2
        for cp in cps[c]:
            cp.wait()

        def row_body(i, carry):
            for j in range(D // 16):
                sl = pl.ds(j * 16, 16)
                acc[b][i, sl] = acc[b][i, sl] + r1[b][i, sl] + r2[b][i, sl]
            return carry
        lax.fori_loop(0, chunk, row_body, 0)
        wcp[c] = pltpu.async_copy(acc[b], out_h.at[pl.ds(base + c * chunk, chunk)], ws[b])
        if c + 2 < nchk:
            wcp[c].wait()
            start(c + 2)
    wcp[nchk - 2].wait()
    wcp[nchk - 1].wait()


def _combine(shared, yg, p1, p2):
    mesh = plsc.VectorSubcoreMesh(core_axis_name="c", subcore_axis_name="s")
    fn = pl.kernel(
        _combine_body,
        mesh=mesh,
        compiler_params=pltpu.CompilerParams(needs_layout_passes=False),
        out_type=jax.ShapeDtypeStruct((T, D), jnp.float32),
        scratch_types=(
            [pltpu.VMEM((16,), jnp.int32) for _ in range(4)]
            + [pltpu.VMEM((16, D), jnp.float32) for _ in range(6)]
            + [pltpu.SemaphoreType.DMA for _ in range(8)]
        ),
    )
    return fn(shared, yg, p1, p2)


def kernel(x, s_gate, s_up, s_down, e_gate, e_up, e_down, router_w, expert_bias):
    B_, S_, D_ = x.shape
    xf = x.reshape(B_ * S_, D_)

    p1, p2, g1, g2, counts, loss, be, xt, used = _router(xf, router_w, expert_bias)
    inv_tok, inv_gate = _build_inv(p1, p2, g1, g2)
    xg = _gather_rows(inv_tok, used, xt)
    shared = _shared_ffn(xf, s_gate.astype(jnp.bfloat16),
                         s_up.astype(jnp.bfloat16), s_down.astype(jnp.bfloat16))
    yg = _expert_ffn(be.reshape(32)[:G], xg, inv_gate,
                     e_gate.astype(jnp.bfloat16), e_up.astype(jnp.bfloat16),
                     e_down.astype(jnp.bfloat16))
    out = _combine(shared, yg, p1, p2)

    output = out.reshape(B_, S_, D_)
    return (output, loss.reshape(()), counts.reshape(E))
